# baseline (speedup 1.0000x reference)
import functools

import jax
import jax.numpy as jnp
from jax import lax
from jax.experimental import pallas as pl
from jax.experimental.pallas import tpu as pltpu
from jax.experimental.pallas import tpu_sc as plsc

BATCH = 4096
D = 64
L = 16
NC = 2
NS = 16
NW = NC * NS
BPW = BATCH // NW
GROUPS = BPW // L
MARGIN = 1.0

_MESH = plsc.VectorSubcoreMesh(core_axis_name="c", subcore_axis_name="s")


@functools.partial(
    pl.kernel,
    out_type=jax.ShapeDtypeStruct((NW, L), jnp.float32),
    mesh=_MESH,
    compiler_params=pltpu.CompilerParams(
        needs_layout_passes=False, use_tc_tiling_on_sc=False),
    scratch_types=[
        pltpu.VMEM((BPW,), jnp.int32),
        pltpu.VMEM((BPW,), jnp.int32),
        pltpu.VMEM((BPW,), jnp.int32),
        pltpu.VMEM((BPW,), jnp.int32),
        pltpu.VMEM((BPW,), jnp.int32),
        pltpu.VMEM((BPW,), jnp.int32),
        pltpu.VMEM((BPW, D), jnp.float32),
        pltpu.VMEM((BPW, D), jnp.float32),
        pltpu.VMEM((BPW, D), jnp.float32),
        pltpu.VMEM((BPW, D), jnp.float32),
        pltpu.VMEM((BPW, D), jnp.float32),
        pltpu.VMEM((BPW, D), jnp.float32),
        pltpu.VMEM((L,), jnp.float32),
        pltpu.SemaphoreType.DMA,
    ],
)
def _transe_sc(ph_h, pt_h, pr_h, nh_h, nt_h, nr_h, ent_h, rel_h, out_h,
               ph_i, pt_i, pr_i, nh_i, nt_i, nr_i,
               ph_r, pt_r, pr_r, nh_r, nt_r, nr_r,
               res_v, sem):
    wid = lax.axis_index("s") * NC + lax.axis_index("c")
    base = wid * BPW

    pltpu.sync_copy(ph_h.at[pl.ds(base, BPW)], ph_i)
    pltpu.sync_copy(pt_h.at[pl.ds(base, BPW)], pt_i)
    pltpu.sync_copy(pr_h.at[pl.ds(base, BPW)], pr_i)
    pltpu.sync_copy(nh_h.at[pl.ds(base, BPW)], nh_i)
    pltpu.sync_copy(nt_h.at[pl.ds(base, BPW)], nt_i)
    pltpu.sync_copy(nr_h.at[pl.ds(base, BPW)], nr_i)

    c1 = pltpu.async_copy(ent_h.at[ph_i], ph_r, sem)
    c2 = pltpu.async_copy(ent_h.at[pt_i], pt_r, sem)
    c3 = pltpu.async_copy(rel_h.at[pr_i], pr_r, sem)
    c4 = pltpu.async_copy(ent_h.at[nh_i], nh_r, sem)
    c5 = pltpu.async_copy(ent_h.at[nt_i], nt_r, sem)
    c6 = pltpu.async_copy(rel_h.at[nr_i], nr_r, sem)
    c1.wait()
    c2.wait()
    c3.wait()
    c4.wait()
    c5.wait()
    c6.wait()

    lanes = lax.iota(jnp.int32, 16)

    def triple_body(i, tot):
        accp = jnp.zeros((L,), jnp.float32)
        accn = jnp.zeros((L,), jnp.float32)
        for c in range(D // L):
            sl = pl.ds(c * L, L)
            accp = accp + jnp.abs(ph_r[i, sl] - pt_r[i, sl] + pr_r[i, sl])
            accn = accn + jnp.abs(nh_r[i, sl] - nt_r[i, sl] + nr_r[i, sl])
        p = jnp.sum(accp)
        n = jnp.sum(accn)
        return tot + jnp.maximum(p - n + MARGIN, 0.0)

    tot = lax.fori_loop(0, BPW, triple_body, jnp.float32(0.0))
    res_v[...] = jnp.where(lanes == 0, tot, 0.0)
    pltpu.sync_copy(res_v, out_h.at[wid])


def kernel(pos_h, pos_t, pos_r, neg_h, neg_t, neg_r, ent_embeddings, rel_embeddings):
    idx = [x.reshape(-1).astype(jnp.int32)
           for x in (pos_h, pos_t, pos_r, neg_h, neg_t, neg_r)]
    partials = _transe_sc(*idx, ent_embeddings, rel_embeddings)
    return jnp.sum(partials)
